# Initial kernel scaffold; baseline (speedup 1.0000x reference)
import jax
import jax.numpy as jnp
from jax.experimental import pallas as pl


def kernel(x, edge_index, batch, W_in, b_in, W, b, gamma, beta):
    raise NotImplementedError("write your pallas kernel here")


# speedup vs baseline: 2.6620x; 2.6620x over previous
import functools

import jax
import jax.numpy as jnp
from jax import lax
from jax.experimental import pallas as pl
from jax.experimental.pallas import tpu as pltpu
from jax.experimental.pallas import tpu_sc as plsc

N = 10000
H = 128
E = 320000
G = 64
L = 3

NC = 2
NS = 16
NTILES = NC * NS
CHUNK = 64
NCHUNK = 160
EPAD = NTILES * NCHUNK * CHUNK
NPAD = 10240
ZROWS = NPAD // NS
RCHUNK = 128


def _sc_aggregate_body(rh_hbm, srcv_hbm, dstv_hbm, out_hbm,
                       srcv, dstv, rows, aggr, sem0, sem1):
    c = lax.axis_index("c")
    s = lax.axis_index("s")

    pltpu.sync_copy(srcv_hbm.at[c, s], srcv)
    pltpu.sync_copy(dstv_hbm.at[c, s], dstv)

    z16 = jnp.zeros((16,), jnp.float32)

    def _zero_row(i, carry):
        for k in range(H // 16):
            rows[0, i, pl.ds(k * 16, 16)] = z16
        return carry

    lax.fori_loop(0, CHUNK, _zero_row, 0)

    def _zero_slab(k, carry):
        pltpu.sync_copy(rows.at[0], aggr.at[pl.ds(s * ZROWS + k * CHUNK, CHUNK)])
        return carry

    lax.fori_loop(0, ZROWS // CHUNK, _zero_slab, 0)
    plsc.subcore_barrier()

    def _round(g, carry):
        d0 = pltpu.async_copy(rh_hbm.at[srcv.at[g, pl.ds(0, CHUNK)]], rows.at[0], sem0)
        d1 = pltpu.async_copy(rh_hbm.at[srcv.at[g, pl.ds(CHUNK, CHUNK)]], rows.at[1], sem1)
        d0.wait()
        pltpu.sync_copy(rows.at[0], aggr.at[dstv.at[2 * g]], add=True)
        d1.wait()
        pltpu.sync_copy(rows.at[1], aggr.at[dstv.at[2 * g + 1]], add=True)
        return carry

    lax.fori_loop(0, NCHUNK // 2, _round, 0)
    plsc.subcore_barrier()

    for k in range(ZROWS // RCHUNK):
        r0 = s * ZROWS + k * RCHUNK
        pltpu.sync_copy(aggr.at[pl.ds(r0, RCHUNK)], out_hbm.at[c, pl.ds(r0, RCHUNK)])


@functools.cache
def _sc_aggregate():
    return pl.kernel(
        _sc_aggregate_body,
        out_type=jax.ShapeDtypeStruct((NC, NPAD, H), jnp.float32),
        mesh=plsc.VectorSubcoreMesh(core_axis_name="c", subcore_axis_name="s",
                                    num_cores=NC, num_subcores=NS),
        scratch_types=[
            pltpu.VMEM((NCHUNK // 2, 2 * CHUNK), jnp.int32),
            pltpu.VMEM((NCHUNK, CHUNK), jnp.int32),
            pltpu.VMEM((2, CHUNK, H), jnp.float32),
            pltpu.VMEM_SHARED((NPAD, H), jnp.float32),
            pltpu.SemaphoreType.DMA,
            pltpu.SemaphoreType.DMA,
        ],
    )


BLK = 2000


def _inproj_body(x_ref, w_ref, b_ref, h_ref, rh_ref):
    z = jnp.dot(x_ref[...], w_ref[...], preferred_element_type=jnp.float32)
    z = z + b_ref[...]
    hz = z * jax.nn.sigmoid(z)
    h_ref[...] = hz
    rh_ref[...] = jnp.maximum(hz, 0.0)


_inproj = pl.pallas_call(
    _inproj_body,
    grid=(N // BLK,),
    in_specs=[
        pl.BlockSpec((BLK, H), lambda i: (i, 0)),
        pl.BlockSpec((H, H), lambda i: (0, 0)),
        pl.BlockSpec((1, H), lambda i: (0, 0)),
    ],
    out_specs=[pl.BlockSpec((BLK, H), lambda i: (i, 0))] * 2,
    out_shape=[jax.ShapeDtypeStruct((N, H), jnp.float32)] * 2,
)


def _layer_body(h_ref, p_ref, w_ref, b_ref, ga_ref, be_ref, out_ref, rh_ref):
    h0 = h_ref[...]
    p = p_ref[...]
    t = h0 + p[0] + p[1]
    z = jnp.dot(t, w_ref[...], preferred_element_type=jnp.float32)
    z = z + b_ref[...]
    mu = jnp.mean(z, axis=-1, keepdims=True)
    zc = z - mu
    var = jnp.mean(zc * zc, axis=-1, keepdims=True)
    zn = zc * lax.rsqrt(var + 1e-5) * ga_ref[...] + be_ref[...]
    zs = zn * jax.nn.sigmoid(zn)
    hn = zs + h0
    out_ref[...] = hn
    rh_ref[...] = jnp.maximum(hn, 0.0)


_layer = pl.pallas_call(
    _layer_body,
    grid=(N // BLK,),
    in_specs=[
        pl.BlockSpec((BLK, H), lambda i: (i, 0)),
        pl.BlockSpec((NC, BLK, H), lambda i: (0, i, 0)),
        pl.BlockSpec((H, H), lambda i: (0, 0)),
        pl.BlockSpec((1, H), lambda i: (0, 0)),
        pl.BlockSpec((1, H), lambda i: (0, 0)),
        pl.BlockSpec((1, H), lambda i: (0, 0)),
    ],
    out_specs=[pl.BlockSpec((BLK, H), lambda i: (i, 0))] * 2,
    out_shape=[jax.ShapeDtypeStruct((N, H), jnp.float32)] * 2,
)


def _pool_body(h_ref, batch_ref, out_ref):
    h = h_ref[...]
    bid = batch_ref[...]
    gids = lax.broadcasted_iota(jnp.int32, (N, G), 1)
    m = (bid == gids).astype(jnp.float32)
    sums = lax.dot_general(m, h, (((0,), (0,)), ((), ())),
                           preferred_element_type=jnp.float32)
    ones = jnp.ones((N, 1), jnp.float32)
    counts = lax.dot_general(m, ones, (((0,), (0,)), ((), ())),
                             preferred_element_type=jnp.float32)
    out_ref[...] = sums / jnp.maximum(counts, 1.0)


_pool = pl.pallas_call(
    _pool_body,
    out_shape=jax.ShapeDtypeStruct((G, H), jnp.float32),
)


def kernel(x, edge_index, batch, W_in, b_in, W, b, gamma, beta):
    src = edge_index[0]
    dst = edge_index[1]
    pad = EPAD - E
    src_p = jnp.concatenate([src, jnp.zeros((pad,), jnp.int32)])
    dst_p = jnp.concatenate([dst, jnp.full((pad,), N, jnp.int32)])
    srcv = src_p.reshape(NC, NS, NCHUNK // 2, 2 * CHUNK)
    dstv = dst_p.reshape(NC, NS, NCHUNK, CHUNK)

    h, rh = _inproj(x, W_in, b_in.reshape(1, H))
    for l in range(L):
        parts = _sc_aggregate()(rh, srcv, dstv)
        h, rh = _layer(h, parts, W[l], b[l].reshape(1, H),
                       gamma[l].reshape(1, H), beta[l].reshape(1, H))
    return _pool(h, batch.reshape(N, 1))


# speedup vs baseline: 3.0043x; 1.1286x over previous
import functools

import jax
import jax.numpy as jnp
from jax import lax
from jax.experimental import pallas as pl
from jax.experimental.pallas import tpu as pltpu
from jax.experimental.pallas import tpu_sc as plsc

N = 10000
H = 128
E = 320000
G = 64
L = 3

NC = 2
NS = 16
NTILES = NC * NS
CHUNK = 64
NCHUNK = 160
EPAD = NTILES * NCHUNK * CHUNK
NPAD = 10240
ZROWS = NPAD // NS
RCHUNK = 128


def _sc_aggregate_body(rh_hbm, srcv_hbm, dstv_hbm, out_hbm,
                       srcv, dstv, rows, aggr, sem0, sem1):
    c = lax.axis_index("c")
    s = lax.axis_index("s")

    pltpu.sync_copy(srcv_hbm.at[c, s], srcv)
    pltpu.sync_copy(dstv_hbm.at[c, s], dstv)

    z16 = jnp.zeros((16,), jnp.float32)

    def _zero_row(i, carry):
        for k in range(H // 16):
            rows[0, i, pl.ds(k * 16, 16)] = z16
        return carry

    lax.fori_loop(0, CHUNK, _zero_row, 0)

    def _zero_slab(k, carry):
        pltpu.sync_copy(rows.at[0], aggr.at[pl.ds(s * ZROWS + k * CHUNK, CHUNK)])
        return carry

    lax.fori_loop(0, ZROWS // CHUNK, _zero_slab, 0)
    plsc.subcore_barrier()

    def _gather(g, half, buf, sem):
        pltpu.async_copy(
            rh_hbm.at[srcv.at[g, pl.ds(half * CHUNK, CHUNK)]], buf, sem)

    def _wait(buf, sem):
        pltpu.make_async_copy(
            rh_hbm.at[srcv.at[0, pl.ds(0, CHUNK)]], buf, sem).wait()

    _gather(0, 0, rows.at[0], sem0)
    _gather(0, 1, rows.at[1], sem1)

    def _round(g, carry):
        _wait(rows.at[0], sem0)
        pltpu.sync_copy(rows.at[0], aggr.at[dstv.at[2 * g]], add=True)

        @pl.when(g < NCHUNK // 2 - 1)
        def _():
            _gather(g + 1, 0, rows.at[0], sem0)

        _wait(rows.at[1], sem1)
        pltpu.sync_copy(rows.at[1], aggr.at[dstv.at[2 * g + 1]], add=True)

        @pl.when(g < NCHUNK // 2 - 1)
        def _():
            _gather(g + 1, 1, rows.at[1], sem1)

        return carry

    lax.fori_loop(0, NCHUNK // 2, _round, 0)
    plsc.subcore_barrier()

    for k in range(ZROWS // RCHUNK):
        r0 = s * ZROWS + k * RCHUNK
        pltpu.sync_copy(aggr.at[pl.ds(r0, RCHUNK)], out_hbm.at[c, pl.ds(r0, RCHUNK)])


@functools.cache
def _sc_aggregate():
    return pl.kernel(
        _sc_aggregate_body,
        out_type=jax.ShapeDtypeStruct((NC, NPAD, H), jnp.float32),
        mesh=plsc.VectorSubcoreMesh(core_axis_name="c", subcore_axis_name="s",
                                    num_cores=NC, num_subcores=NS),
        scratch_types=[
            pltpu.VMEM((NCHUNK // 2, 2 * CHUNK), jnp.int32),
            pltpu.VMEM((NCHUNK, CHUNK), jnp.int32),
            pltpu.VMEM((2, CHUNK, H), jnp.float32),
            pltpu.VMEM_SHARED((NPAD, H), jnp.float32),
            pltpu.SemaphoreType.DMA,
            pltpu.SemaphoreType.DMA,
        ],
    )


BLK = 2000


def _inproj_body(x_ref, w_ref, b_ref, h_ref, rh_ref):
    z = jnp.dot(x_ref[...], w_ref[...], preferred_element_type=jnp.float32)
    z = z + b_ref[...]
    hz = z * jax.nn.sigmoid(z)
    h_ref[...] = hz
    rh_ref[...] = jnp.maximum(hz, 0.0)


_inproj = pl.pallas_call(
    _inproj_body,
    grid=(N // BLK,),
    in_specs=[
        pl.BlockSpec((BLK, H), lambda i: (i, 0)),
        pl.BlockSpec((H, H), lambda i: (0, 0)),
        pl.BlockSpec((1, H), lambda i: (0, 0)),
    ],
    out_specs=[pl.BlockSpec((BLK, H), lambda i: (i, 0))] * 2,
    out_shape=[jax.ShapeDtypeStruct((N, H), jnp.float32)] * 2,
)


def _layer_body(h_ref, p_ref, w_ref, b_ref, ga_ref, be_ref, out_ref, rh_ref):
    h0 = h_ref[...]
    p = p_ref[...]
    t = h0 + p[0] + p[1]
    z = jnp.dot(t, w_ref[...], preferred_element_type=jnp.float32)
    z = z + b_ref[...]
    mu = jnp.mean(z, axis=-1, keepdims=True)
    zc = z - mu
    var = jnp.mean(zc * zc, axis=-1, keepdims=True)
    zn = zc * lax.rsqrt(var + 1e-5) * ga_ref[...] + be_ref[...]
    zs = zn * jax.nn.sigmoid(zn)
    hn = zs + h0
    out_ref[...] = hn
    rh_ref[...] = jnp.maximum(hn, 0.0)


_layer = pl.pallas_call(
    _layer_body,
    grid=(N // BLK,),
    in_specs=[
        pl.BlockSpec((BLK, H), lambda i: (i, 0)),
        pl.BlockSpec((NC, BLK, H), lambda i: (0, i, 0)),
        pl.BlockSpec((H, H), lambda i: (0, 0)),
        pl.BlockSpec((1, H), lambda i: (0, 0)),
        pl.BlockSpec((1, H), lambda i: (0, 0)),
        pl.BlockSpec((1, H), lambda i: (0, 0)),
    ],
    out_specs=[pl.BlockSpec((BLK, H), lambda i: (i, 0))] * 2,
    out_shape=[jax.ShapeDtypeStruct((N, H), jnp.float32)] * 2,
)


def _pool_body(h_ref, batch_ref, out_ref):
    h = h_ref[...]
    bid = batch_ref[...]
    gids = lax.broadcasted_iota(jnp.int32, (N, G), 1)
    m = (bid == gids).astype(jnp.float32)
    sums = lax.dot_general(m, h, (((0,), (0,)), ((), ())),
                           preferred_element_type=jnp.float32)
    ones = jnp.ones((N, 1), jnp.float32)
    counts = lax.dot_general(m, ones, (((0,), (0,)), ((), ())),
                             preferred_element_type=jnp.float32)
    out_ref[...] = sums / jnp.maximum(counts, 1.0)


_pool = pl.pallas_call(
    _pool_body,
    out_shape=jax.ShapeDtypeStruct((G, H), jnp.float32),
)


def kernel(x, edge_index, batch, W_in, b_in, W, b, gamma, beta):
    src = edge_index[0]
    dst = edge_index[1]
    pad = EPAD - E
    src_p = jnp.concatenate([src, jnp.zeros((pad,), jnp.int32)])
    trash = N + jnp.arange(pad, dtype=jnp.int32) % (NPAD - N)
    dst_p = jnp.concatenate([dst, trash])
    srcv = src_p.reshape(NC, NS, NCHUNK // 2, 2 * CHUNK)
    dstv = dst_p.reshape(NC, NS, NCHUNK, CHUNK)

    h, rh = _inproj(x, W_in, b_in.reshape(1, H))
    for l in range(L):
        parts = _sc_aggregate()(rh, srcv, dstv)
        h, rh = _layer(h, parts, W[l], b[l].reshape(1, H),
                       gamma[l].reshape(1, H), beta[l].reshape(1, H))
    return _pool(h, batch.reshape(N, 1))
